# baseline (device time: 28081 ns/iter reference)
import jax
import jax.numpy as jnp
from jax import lax
from jax.experimental import pallas as pl
from jax.experimental.pallas import tpu as pltpu

N_DEV = 4
EPS = 1e-5
LANES = 128
CHUNK = 512


def kernel(x, gamma):
    m, n_local = x.shape
    n_global = n_local * N_DEV
    sub = m // LANES
    k_chunks = m // CHUNK

    gamma2 = gamma.reshape(1, n_local)

    def body(
        x_hbm, g_ref, out_hbm,
        xv, pre_ref, pcol_ref, comm_ref, outv,
        in_sems, out_sems, send_sems, recv_sems,
    ):
        my = lax.axis_index("i")

        in_copies = []
        for k in range(k_chunks):
            rows = pl.ds(k * CHUNK, CHUNK)
            cp = pltpu.make_async_copy(x_hbm.at[rows, :], xv.at[rows, :],
                                       in_sems.at[k])
            cp.start()
            in_copies.append(cp)
        for k in range(k_chunks):
            in_copies[k].wait()
            rows = pl.ds(k * CHUNK, CHUNK)
            xc = xv[rows, :]
            pcol_ref[rows, :] = jnp.sum(xc * xc, axis=1, keepdims=True)

        ri = lax.broadcasted_iota(jnp.int32, (m, LANES), 0)
        li = lax.broadcasted_iota(jnp.int32, (m, LANES), 1)
        mask = (jnp.bitwise_and(ri, LANES - 1) == li).astype(jnp.float32)
        si_r = lax.broadcasted_iota(jnp.int32, (sub, m), 0)
        ri_r = lax.broadcasted_iota(jnp.int32, (sub, m), 1)
        lt = (jnp.right_shift(ri_r, 7) == si_r).astype(jnp.float32)

        packed = jnp.dot(lt, pcol_ref[:, :] * mask,
                         preferred_element_type=jnp.float32)
        comm_ref[pl.ds(my, 1)] = packed[None]

        sends = []
        for d in range(1, N_DEV):
            dst = lax.rem(my + d, N_DEV)
            rdma = pltpu.make_async_remote_copy(
                src_ref=comm_ref.at[my],
                dst_ref=comm_ref.at[my],
                send_sem=send_sems.at[d - 1],
                recv_sem=recv_sems.at[my],
                device_id=(dst,),
                device_id_type=pl.DeviceIdType.MESH,
            )
            rdma.start()
            sends.append(rdma)

        gf = g_ref[:, :].astype(jnp.float32)
        pre_ref[:, :] = xv[:, :] * gf
        ri_c = lax.broadcasted_iota(jnp.int32, (m, sub), 0)
        si_c = lax.broadcasted_iota(jnp.int32, (m, sub), 1)
        lsel = (jnp.right_shift(ri_c, 7) == si_c).astype(jnp.float32)

        for d in range(1, N_DEV):
            src = lax.rem(my + d, N_DEV)
            recv = pltpu.make_async_remote_copy(
                src_ref=comm_ref.at[src],
                dst_ref=comm_ref.at[src],
                send_sem=send_sems.at[d - 1],
                recv_sem=recv_sems.at[src],
                device_id=(my,),
                device_id_type=pl.DeviceIdType.MESH,
            )
            recv.wait_recv()

        total = comm_ref[0] + comm_ref[1] + comm_ref[2] + comm_ref[3]
        inv_packed = lax.rsqrt(total / n_global + EPS)
        a = jnp.dot(lsel, inv_packed, preferred_element_type=jnp.float32)
        inv_col = jnp.sum(a * mask, axis=1, keepdims=True)

        out_copies = [None, None]
        for k in range(k_chunks):
            slot = k % 2
            if out_copies[slot] is not None:
                out_copies[slot].wait()
            rows = pl.ds(k * CHUNK, CHUNK)
            outv[slot] = (
                pre_ref[rows, :] * inv_col[k * CHUNK:(k + 1) * CHUNK, :]
            ).astype(jnp.bfloat16)
            cp = pltpu.make_async_copy(outv.at[slot], out_hbm.at[rows, :],
                                       out_sems.at[slot])
            cp.start()
            out_copies[slot] = cp
        for cp in out_copies:
            cp.wait()

        for rdma in sends:
            rdma.wait_send()

    return pl.pallas_call(
        body,
        out_shape=jax.ShapeDtypeStruct((m, n_local), jnp.bfloat16),
        in_specs=[
            pl.BlockSpec(memory_space=pl.ANY),
            pl.BlockSpec(memory_space=pltpu.VMEM),
        ],
        out_specs=pl.BlockSpec(memory_space=pl.ANY),
        scratch_shapes=[
            pltpu.VMEM((m, n_local), jnp.float32),
            pltpu.VMEM((m, n_local), jnp.float32),
            pltpu.VMEM((m, 1), jnp.float32),
            pltpu.VMEM((N_DEV, sub, LANES), jnp.float32),
            pltpu.VMEM((2, CHUNK, n_local), jnp.bfloat16),
            pltpu.SemaphoreType.DMA((8,)),
            pltpu.SemaphoreType.DMA((2,)),
            pltpu.SemaphoreType.DMA((N_DEV - 1,)),
            pltpu.SemaphoreType.DMA((N_DEV,)),
        ],
        compiler_params=pltpu.CompilerParams(
            vmem_limit_bytes=64 * 1024 * 1024,
        ),
    )(x, gamma2)


# device time: 24680 ns/iter; 1.1378x vs baseline; 1.1378x over previous
import jax
import jax.numpy as jnp
from jax import lax
from jax.experimental import pallas as pl
from jax.experimental.pallas import tpu as pltpu

N_DEV = 4
EPS = 1e-5
LANES = 128
CHUNK = 1024


def kernel(x, gamma):
    m, n_local = x.shape
    n_global = n_local * N_DEV
    kc = m // CHUNK
    csub = CHUNK // LANES

    gamma2 = gamma.reshape(1, n_local)

    def body(
        x_hbm, g_ref, out_hbm,
        xv, comm_ref, outv,
        in_sems, out_sems, send_sems, recv_sems,
    ):
        my = lax.axis_index("i")

        ri = lax.broadcasted_iota(jnp.int32, (CHUNK, LANES), 0)
        li = lax.broadcasted_iota(jnp.int32, (CHUNK, LANES), 1)
        mask = (jnp.bitwise_and(ri, LANES - 1) == li).astype(jnp.float32)
        si_r = lax.broadcasted_iota(jnp.int32, (csub, CHUNK), 0)
        ri_r = lax.broadcasted_iota(jnp.int32, (csub, CHUNK), 1)
        lt = (jnp.right_shift(ri_r, 7) == si_r).astype(jnp.float32)
        ri_c = lax.broadcasted_iota(jnp.int32, (CHUNK, csub), 0)
        si_c = lax.broadcasted_iota(jnp.int32, (CHUNK, csub), 1)
        lsel = (jnp.right_shift(ri_c, 7) == si_c).astype(jnp.float32)
        gf = g_ref[:, :].astype(jnp.float32)

        in_copies = []
        for k in range(kc):
            rows = pl.ds(k * CHUNK, CHUNK)
            cp = pltpu.make_async_copy(x_hbm.at[rows, :], xv.at[rows, :],
                                       in_sems.at[k])
            cp.start()
            in_copies.append(cp)

        sends = []
        for k in range(kc):
            in_copies[k].wait()
            xc = xv[pl.ds(k * CHUNK, CHUNK), :]
            p_col = jnp.sum(xc * xc, axis=1, keepdims=True)
            tile = jnp.dot(lt, p_col * mask,
                           preferred_element_type=jnp.float32)
            comm_ref[my, k] = tile
            for d in range(1, N_DEV):
                dst = lax.rem(my + d, N_DEV)
                rdma = pltpu.make_async_remote_copy(
                    src_ref=comm_ref.at[my, k],
                    dst_ref=comm_ref.at[my, k],
                    send_sem=send_sems.at[d - 1, k],
                    recv_sem=recv_sems.at[my, k],
                    device_id=(dst,),
                    device_id_type=pl.DeviceIdType.MESH,
                )
                rdma.start()
                sends.append(rdma)

        out_copies = [None, None]
        for k in range(kc):
            for d in range(1, N_DEV):
                src = lax.rem(my + d, N_DEV)
                recv = pltpu.make_async_remote_copy(
                    src_ref=comm_ref.at[src, k],
                    dst_ref=comm_ref.at[src, k],
                    send_sem=send_sems.at[d - 1, k],
                    recv_sem=recv_sems.at[src, k],
                    device_id=(my,),
                    device_id_type=pl.DeviceIdType.MESH,
                )
                recv.wait_recv()
            total = (comm_ref[0, k] + comm_ref[1, k]
                     + comm_ref[2, k] + comm_ref[3, k])
            inv_tile = lax.rsqrt(total / n_global + EPS)
            a = jnp.dot(lsel, inv_tile,
                        preferred_element_type=jnp.float32)
            inv_col = jnp.sum(a * mask, axis=1, keepdims=True)

            slot = k % 2
            if out_copies[slot] is not None:
                out_copies[slot].wait()
            rows = pl.ds(k * CHUNK, CHUNK)
            outv[slot] = (xv[rows, :] * gf * inv_col).astype(jnp.bfloat16)
            cp = pltpu.make_async_copy(outv.at[slot], out_hbm.at[rows, :],
                                       out_sems.at[slot])
            cp.start()
            out_copies[slot] = cp
        for cp in out_copies:
            cp.wait()

        for rdma in sends:
            rdma.wait_send()

    return pl.pallas_call(
        body,
        out_shape=jax.ShapeDtypeStruct((m, n_local), jnp.bfloat16),
        in_specs=[
            pl.BlockSpec(memory_space=pl.ANY),
            pl.BlockSpec(memory_space=pltpu.VMEM),
        ],
        out_specs=pl.BlockSpec(memory_space=pl.ANY),
        scratch_shapes=[
            pltpu.VMEM((m, n_local), jnp.float32),
            pltpu.VMEM((N_DEV, kc, csub, LANES), jnp.float32),
            pltpu.VMEM((2, CHUNK, n_local), jnp.bfloat16),
            pltpu.SemaphoreType.DMA((kc,)),
            pltpu.SemaphoreType.DMA((2,)),
            pltpu.SemaphoreType.DMA((N_DEV - 1, kc)),
            pltpu.SemaphoreType.DMA((N_DEV, kc)),
        ],
        compiler_params=pltpu.CompilerParams(
            vmem_limit_bytes=64 * 1024 * 1024,
        ),
    )(x, gamma2)
